# baseline (speedup 1.0000x reference)
import functools

import jax
import jax.numpy as jnp
from jax import lax
from jax.experimental import pallas as pl
from jax.experimental.pallas import tpu as pltpu
from jax.experimental.pallas import tpu_sc as plsc

N_NODES = 10000
N_EDGES = 160000
D = 256
HALF = 128

NUM_CORES = 2
NUM_TILES = 16

CHUNK = 105
NBUF = 3
SECT_CHUNKS = 12
N_SECT = 8
CHUNKS_PER_TILE = SECT_CHUNKS * N_SECT
N_CHUNKS = CHUNKS_PER_TILE * NUM_TILES
E_PAD = N_CHUNKS * CHUNK
ROWS_PER_TILE = 632
N_PAD = ROWS_PER_TILE * NUM_TILES

R_BLK = 2000


def _elu(v):
    return jnp.where(v > 0, v, jnp.exp(v) - 1.0)


def _mm_first_body(x_ref, w_ref, o_ref):
    xb = x_ref[...]
    o_ref[0] = jnp.dot(xb, w_ref[:, :HALF], preferred_element_type=jnp.float32)
    o_ref[1] = jnp.dot(xb, w_ref[:, HALF:], preferred_element_type=jnp.float32)


def _mm_first(x, w):
    grid = (N_NODES // R_BLK,)
    return pl.pallas_call(
        _mm_first_body,
        grid=grid,
        in_specs=[
            pl.BlockSpec((R_BLK, D), lambda i: (i, 0)),
            pl.BlockSpec((D, D), lambda i: (0, 0)),
        ],
        out_specs=pl.BlockSpec((NUM_CORES, R_BLK, HALF), lambda i: (0, i, 0)),
        out_shape=jax.ShapeDtypeStruct((NUM_CORES, N_NODES, HALF), jnp.float32),
    )(x, w)


def _mm_mid_body(y_ref, a_ref, b_ref, w_ref, o_ref):
    h0 = _elu(y_ref[0] + a_ref[0] + b_ref[0])
    h1 = _elu(y_ref[1] + a_ref[1] + b_ref[1])
    r = jnp.dot(h0, w_ref[:HALF, :], preferred_element_type=jnp.float32)
    r = r + jnp.dot(h1, w_ref[HALF:, :], preferred_element_type=jnp.float32)
    o_ref[0] = r[:, :HALF]
    o_ref[1] = r[:, HALF:]


def _mm_mid(y, agg, b2d, w):
    grid = (N_NODES // R_BLK,)
    return pl.pallas_call(
        _mm_mid_body,
        grid=grid,
        in_specs=[
            pl.BlockSpec((NUM_CORES, R_BLK, HALF), lambda i: (0, i, 0)),
            pl.BlockSpec((NUM_CORES, R_BLK, HALF), lambda i: (0, i, 0)),
            pl.BlockSpec((NUM_CORES, 1, HALF), lambda i: (0, 0, 0)),
            pl.BlockSpec((D, D), lambda i: (0, 0)),
        ],
        out_specs=pl.BlockSpec((NUM_CORES, R_BLK, HALF), lambda i: (0, i, 0)),
        out_shape=jax.ShapeDtypeStruct((NUM_CORES, N_NODES, HALF), jnp.float32),
    )(y, agg, b2d, w)


def _final_body(y_ref, a_ref, b_ref, o_ref):
    o_ref[:, :HALF] = y_ref[0] + a_ref[0] + b_ref[0]
    o_ref[:, HALF:] = y_ref[1] + a_ref[1] + b_ref[1]


def _final(y, agg, b2d):
    grid = (N_NODES // R_BLK,)
    return pl.pallas_call(
        _final_body,
        grid=grid,
        in_specs=[
            pl.BlockSpec((NUM_CORES, R_BLK, HALF), lambda i: (0, i, 0)),
            pl.BlockSpec((NUM_CORES, R_BLK, HALF), lambda i: (0, i, 0)),
            pl.BlockSpec((NUM_CORES, 1, HALF), lambda i: (0, 0, 0)),
        ],
        out_specs=pl.BlockSpec((R_BLK, D), lambda i: (i, 0)),
        out_shape=jax.ShapeDtypeStruct((N_NODES, D), jnp.float32),
    )(y, agg, b2d)


def _sc_agg_body(y_hbm, idxc_hbm, zeros_hbm, out_hbm,
                 idx_v, rows_v, acc_sh, *sems):
    gsems = sems[:NBUF]
    ssems = sems[NBUF:]
    c = lax.axis_index("c")
    s = lax.axis_index("s")
    base = s * CHUNKS_PER_TILE

    pltpu.sync_copy(zeros_hbm, acc_sh.at[pl.ds(s * ROWS_PER_TILE, ROWS_PER_TILE)])

    plsc.subcore_barrier()

    def s_wait(b):
        pltpu.make_async_copy(
            rows_v.at[b], acc_sh.at[idx_v.at[b].at[1]], ssems[b]).wait()

    def body(j, carry):
        @pl.when(j > 0)
        def _():
            for b in range(NBUF):
                s_wait(b)
        gathers = [
            pltpu.async_copy(y_hbm.at[idx_v.at[NBUF * j + b].at[0]],
                             rows_v.at[b], gsems[b])
            for b in range(NBUF)
        ]
        for b in range(NBUF):
            gathers[b].wait()
            pltpu.async_copy(rows_v.at[b],
                             acc_sh.at[idx_v.at[NBUF * j + b].at[1]],
                             ssems[b], add=True)
        return carry

    for sec in range(N_SECT):
        if sec > 0:
            for b in range(NBUF):
                s_wait(b)
        pltpu.sync_copy(
            idxc_hbm.at[c].at[pl.ds(base + sec * SECT_CHUNKS, SECT_CHUNKS)],
            idx_v)
        lax.fori_loop(0, SECT_CHUNKS // NBUF, body, 0)

    for b in range(NBUF):
        s_wait(b)

    plsc.subcore_barrier()

    r0 = s * ROWS_PER_TILE
    pltpu.sync_copy(
        acc_sh.at[pl.ds(r0, ROWS_PER_TILE)],
        out_hbm.at[c].at[pl.ds(r0, ROWS_PER_TILE)],
    )


@functools.lru_cache(maxsize=1)
def _make_sc_agg_kernel():
    return pl.kernel(
        _sc_agg_body,
        out_type=jax.ShapeDtypeStruct((NUM_CORES, N_PAD, HALF), jnp.float32),
        mesh=plsc.VectorSubcoreMesh(
            core_axis_name="c", subcore_axis_name="s",
            num_cores=NUM_CORES, num_subcores=NUM_TILES,
        ),
        scratch_types=[
            pltpu.VMEM((SECT_CHUNKS, 2, CHUNK), jnp.int32),
            pltpu.VMEM((NBUF, CHUNK, HALF), jnp.float32),
            pltpu.VMEM_SHARED((N_PAD, HALF), jnp.float32),
        ] + [pltpu.SemaphoreType.DMA] * (2 * NBUF),
    )


def _sc_agg(y, idxc, zeros):
    yf = y.reshape(NUM_CORES * N_NODES, HALF)
    return _make_sc_agg_kernel()(yf, idxc, zeros)


def kernel(x, edge_index, W1, b1, W2, b2, W3, b3):
    ei = edge_index.astype(jnp.int32)
    src = ei[0]
    dst = ei[1]

    pad = E_PAD - N_EDGES
    src_p = jnp.concatenate([src, jnp.zeros((pad,), jnp.int32)])
    dst_p = jnp.concatenate([dst, jnp.full((pad,), N_NODES, jnp.int32)])
    srcc = src_p.reshape(N_CHUNKS, CHUNK)
    dstc = dst_p.reshape(N_CHUNKS, CHUNK)
    idxc = jnp.stack([
        jnp.stack([srcc, dstc], axis=1),
        jnp.stack([srcc + N_NODES, dstc], axis=1),
    ])
    zeros = jnp.zeros((ROWS_PER_TILE, HALF), jnp.float32)

    b1h = b1.reshape(NUM_CORES, 1, HALF)
    b2h = b2.reshape(NUM_CORES, 1, HALF)
    b3h = b3.reshape(NUM_CORES, 1, HALF)

    y1 = _mm_first(x, W1)
    a1 = _sc_agg(y1, idxc, zeros)
    y2 = _mm_mid(y1, a1, b1h, W2)
    a2 = _sc_agg(y2, idxc, zeros)
    y3 = _mm_mid(y2, a2, b2h, W3)
    a3 = _sc_agg(y3, idxc, zeros)
    return _final(y3, a3, b3h)


# speedup vs baseline: 1.0081x; 1.0081x over previous
import functools

import jax
import jax.numpy as jnp
from jax import lax
from jax.experimental import pallas as pl
from jax.experimental.pallas import tpu as pltpu
from jax.experimental.pallas import tpu_sc as plsc

N_NODES = 10000
N_EDGES = 160000
D = 256
HALF = 128

NUM_CORES = 2
NUM_TILES = 16

CHUNK = 105
NBUF = 3
SECT_CHUNKS = 24
N_SECT = 4
CHUNKS_PER_TILE = SECT_CHUNKS * N_SECT
N_CHUNKS = CHUNKS_PER_TILE * NUM_TILES
E_PAD = N_CHUNKS * CHUNK
ROWS_PER_TILE = 632
N_PAD = ROWS_PER_TILE * NUM_TILES

R_BLK = 2000


def _elu(v):
    return jnp.where(v > 0, v, jnp.exp(v) - 1.0)


def _mm_first_body(x_ref, w_ref, o_ref):
    xb = x_ref[...]
    o_ref[0] = jnp.dot(xb, w_ref[:, :HALF], preferred_element_type=jnp.float32)
    o_ref[1] = jnp.dot(xb, w_ref[:, HALF:], preferred_element_type=jnp.float32)


def _mm_first(x, w):
    grid = (N_NODES // R_BLK,)
    return pl.pallas_call(
        _mm_first_body,
        grid=grid,
        in_specs=[
            pl.BlockSpec((R_BLK, D), lambda i: (i, 0)),
            pl.BlockSpec((D, D), lambda i: (0, 0)),
        ],
        out_specs=pl.BlockSpec((NUM_CORES, R_BLK, HALF), lambda i: (0, i, 0)),
        out_shape=jax.ShapeDtypeStruct((NUM_CORES, N_NODES, HALF), jnp.float32),
    )(x, w)


def _mm_mid_body(y_ref, a_ref, b_ref, w_ref, o_ref):
    h0 = _elu(y_ref[0] + a_ref[0] + b_ref[0])
    h1 = _elu(y_ref[1] + a_ref[1] + b_ref[1])
    r = jnp.dot(h0, w_ref[:HALF, :], preferred_element_type=jnp.float32)
    r = r + jnp.dot(h1, w_ref[HALF:, :], preferred_element_type=jnp.float32)
    o_ref[0] = r[:, :HALF]
    o_ref[1] = r[:, HALF:]


def _mm_mid(y, agg, b2d, w):
    grid = (N_NODES // R_BLK,)
    return pl.pallas_call(
        _mm_mid_body,
        grid=grid,
        in_specs=[
            pl.BlockSpec((NUM_CORES, R_BLK, HALF), lambda i: (0, i, 0)),
            pl.BlockSpec((NUM_CORES, R_BLK, HALF), lambda i: (0, i, 0)),
            pl.BlockSpec((NUM_CORES, 1, HALF), lambda i: (0, 0, 0)),
            pl.BlockSpec((D, D), lambda i: (0, 0)),
        ],
        out_specs=pl.BlockSpec((NUM_CORES, R_BLK, HALF), lambda i: (0, i, 0)),
        out_shape=jax.ShapeDtypeStruct((NUM_CORES, N_NODES, HALF), jnp.float32),
    )(y, agg, b2d, w)


def _final_body(y_ref, a_ref, b_ref, o_ref):
    o_ref[:, :HALF] = y_ref[0] + a_ref[0] + b_ref[0]
    o_ref[:, HALF:] = y_ref[1] + a_ref[1] + b_ref[1]


def _final(y, agg, b2d):
    grid = (N_NODES // R_BLK,)
    return pl.pallas_call(
        _final_body,
        grid=grid,
        in_specs=[
            pl.BlockSpec((NUM_CORES, R_BLK, HALF), lambda i: (0, i, 0)),
            pl.BlockSpec((NUM_CORES, R_BLK, HALF), lambda i: (0, i, 0)),
            pl.BlockSpec((NUM_CORES, 1, HALF), lambda i: (0, 0, 0)),
        ],
        out_specs=pl.BlockSpec((R_BLK, D), lambda i: (i, 0)),
        out_shape=jax.ShapeDtypeStruct((N_NODES, D), jnp.float32),
    )(y, agg, b2d)


def _sc_agg_body(y_hbm, idxc_hbm, zeros_hbm, out_hbm,
                 idx_v, rows_v, acc_sh, *sems):
    gsems = sems[:NBUF]
    ssems = sems[NBUF:]
    c = lax.axis_index("c")
    s = lax.axis_index("s")
    base = s * CHUNKS_PER_TILE

    pltpu.sync_copy(zeros_hbm, acc_sh.at[pl.ds(s * ROWS_PER_TILE, ROWS_PER_TILE)])

    plsc.subcore_barrier()

    def s_wait(b):
        pltpu.make_async_copy(
            rows_v.at[b], acc_sh.at[idx_v.at[b].at[1]], ssems[b]).wait()

    def body(j, carry):
        @pl.when(j > 0)
        def _():
            for b in range(NBUF):
                s_wait(b)
        gathers = [
            pltpu.async_copy(y_hbm.at[idx_v.at[NBUF * j + b].at[0]],
                             rows_v.at[b], gsems[b])
            for b in range(NBUF)
        ]
        for b in range(NBUF):
            gathers[b].wait()
            pltpu.async_copy(rows_v.at[b],
                             acc_sh.at[idx_v.at[NBUF * j + b].at[1]],
                             ssems[b], add=True)
        return carry

    for sec in range(N_SECT):
        if sec > 0:
            for b in range(NBUF):
                s_wait(b)
        pltpu.sync_copy(
            idxc_hbm.at[c].at[pl.ds(base + sec * SECT_CHUNKS, SECT_CHUNKS)],
            idx_v)
        lax.fori_loop(0, SECT_CHUNKS // NBUF, body, 0)

    for b in range(NBUF):
        s_wait(b)

    plsc.subcore_barrier()

    r0 = s * ROWS_PER_TILE
    pltpu.sync_copy(
        acc_sh.at[pl.ds(r0, ROWS_PER_TILE)],
        out_hbm.at[c].at[pl.ds(r0, ROWS_PER_TILE)],
    )


@functools.lru_cache(maxsize=1)
def _make_sc_agg_kernel():
    return pl.kernel(
        _sc_agg_body,
        out_type=jax.ShapeDtypeStruct((NUM_CORES, N_PAD, HALF), jnp.float32),
        mesh=plsc.VectorSubcoreMesh(
            core_axis_name="c", subcore_axis_name="s",
            num_cores=NUM_CORES, num_subcores=NUM_TILES,
        ),
        scratch_types=[
            pltpu.VMEM((SECT_CHUNKS, 2, CHUNK), jnp.int32),
            pltpu.VMEM((NBUF, CHUNK, HALF), jnp.float32),
            pltpu.VMEM_SHARED((N_PAD, HALF), jnp.float32),
        ] + [pltpu.SemaphoreType.DMA] * (2 * NBUF),
    )


def _sc_agg(y, idxc, zeros):
    yf = y.reshape(NUM_CORES * N_NODES, HALF)
    return _make_sc_agg_kernel()(yf, idxc, zeros)


def kernel(x, edge_index, W1, b1, W2, b2, W3, b3):
    ei = edge_index.astype(jnp.int32)
    src = ei[0]
    dst = ei[1]

    pad = E_PAD - N_EDGES
    src_p = jnp.concatenate([src, jnp.zeros((pad,), jnp.int32)])
    dst_p = jnp.concatenate([dst, jnp.full((pad,), N_NODES, jnp.int32)])
    srcc = src_p.reshape(N_CHUNKS, CHUNK)
    dstc = dst_p.reshape(N_CHUNKS, CHUNK)
    idxc = jnp.stack([
        jnp.stack([srcc, dstc], axis=1),
        jnp.stack([srcc + N_NODES, dstc], axis=1),
    ])
    zeros = jnp.zeros((ROWS_PER_TILE, HALF), jnp.float32)

    b1h = b1.reshape(NUM_CORES, 1, HALF)
    b2h = b2.reshape(NUM_CORES, 1, HALF)
    b3h = b3.reshape(NUM_CORES, 1, HALF)

    y1 = _mm_first(x, W1)
    a1 = _sc_agg(y1, idxc, zeros)
    y2 = _mm_mid(y1, a1, b1h, W2)
    a2 = _sc_agg(y2, idxc, zeros)
    y3 = _mm_mid(y2, a2, b2h, W3)
    a3 = _sc_agg(y3, idxc, zeros)
    return _final(y3, a3, b3h)


# speedup vs baseline: 1.3570x; 1.3461x over previous
import functools

import jax
import jax.numpy as jnp
from jax import lax
from jax.experimental import pallas as pl
from jax.experimental.pallas import tpu as pltpu
from jax.experimental.pallas import tpu_sc as plsc

N_NODES = 10000
N_EDGES = 160000
D = 256
HALF = 128

NUM_CORES = 2
NUM_TILES = 16

CHUNK = 125
NBUF = 2
CHUNKS_PER_TILE = 80
N_CHUNKS = CHUNKS_PER_TILE * NUM_TILES
E_PAD = N_CHUNKS * CHUNK
HALF_CHUNKS = CHUNKS_PER_TILE // 2
ROWS_PER_TILE = 632
N_PAD = ROWS_PER_TILE * NUM_TILES

R_BLK = 2000


def _elu(v):
    return jnp.where(v > 0, v, jnp.exp(v) - 1.0)


def _mm_first_body(x_ref, w_ref, o_ref):
    xb = x_ref[...]
    o_ref[0] = jnp.dot(xb, w_ref[:, :HALF], preferred_element_type=jnp.float32)
    o_ref[1] = jnp.dot(xb, w_ref[:, HALF:], preferred_element_type=jnp.float32)


def _mm_first(x, w):
    grid = (N_NODES // R_BLK,)
    return pl.pallas_call(
        _mm_first_body,
        grid=grid,
        in_specs=[
            pl.BlockSpec((R_BLK, D), lambda i: (i, 0)),
            pl.BlockSpec((D, D), lambda i: (0, 0)),
        ],
        out_specs=pl.BlockSpec((NUM_CORES, R_BLK, HALF), lambda i: (0, i, 0)),
        out_shape=jax.ShapeDtypeStruct((NUM_CORES, N_NODES, HALF), jnp.float32),
    )(x, w)


def _mm_mid_body(y_ref, a_ref, b_ref, w_ref, o_ref):
    h0 = _elu(y_ref[0] + a_ref[0] + b_ref[0])
    h1 = _elu(y_ref[1] + a_ref[1] + b_ref[1])
    r = jnp.dot(h0, w_ref[:HALF, :], preferred_element_type=jnp.float32)
    r = r + jnp.dot(h1, w_ref[HALF:, :], preferred_element_type=jnp.float32)
    o_ref[0] = r[:, :HALF]
    o_ref[1] = r[:, HALF:]


def _mm_mid(y, agg, b2d, w):
    grid = (N_NODES // R_BLK,)
    return pl.pallas_call(
        _mm_mid_body,
        grid=grid,
        in_specs=[
            pl.BlockSpec((NUM_CORES, R_BLK, HALF), lambda i: (0, i, 0)),
            pl.BlockSpec((NUM_CORES, R_BLK, HALF), lambda i: (0, i, 0)),
            pl.BlockSpec((NUM_CORES, 1, HALF), lambda i: (0, 0, 0)),
            pl.BlockSpec((D, D), lambda i: (0, 0)),
        ],
        out_specs=pl.BlockSpec((NUM_CORES, R_BLK, HALF), lambda i: (0, i, 0)),
        out_shape=jax.ShapeDtypeStruct((NUM_CORES, N_NODES, HALF), jnp.float32),
    )(y, agg, b2d, w)


def _final_body(y_ref, a_ref, b_ref, o_ref):
    o_ref[:, :HALF] = y_ref[0] + a_ref[0] + b_ref[0]
    o_ref[:, HALF:] = y_ref[1] + a_ref[1] + b_ref[1]


def _final(y, agg, b2d):
    grid = (N_NODES // R_BLK,)
    return pl.pallas_call(
        _final_body,
        grid=grid,
        in_specs=[
            pl.BlockSpec((NUM_CORES, R_BLK, HALF), lambda i: (0, i, 0)),
            pl.BlockSpec((NUM_CORES, R_BLK, HALF), lambda i: (0, i, 0)),
            pl.BlockSpec((NUM_CORES, 1, HALF), lambda i: (0, 0, 0)),
        ],
        out_specs=pl.BlockSpec((R_BLK, D), lambda i: (i, 0)),
        out_shape=jax.ShapeDtypeStruct((N_NODES, D), jnp.float32),
    )(y, agg, b2d)


def _sc_agg_body(y_hbm, idxc_hbm, zeros_hbm, out_hbm,
                 idx_v, rows_v, acc_sh, *sems):
    gsems = sems[0:2]
    ssems = sems[2:4]
    c = lax.axis_index("c")
    s = lax.axis_index("s")
    base = s * CHUNKS_PER_TILE

    pltpu.sync_copy(zeros_hbm, acc_sh.at[pl.ds(s * ROWS_PER_TILE, ROWS_PER_TILE)])

    pltpu.sync_copy(idxc_hbm.at[c].at[pl.ds(base, HALF_CHUNKS)], idx_v)

    plsc.subcore_barrier()

    def s_wait(b):
        pltpu.make_async_copy(
            rows_v.at[b], acc_sh.at[idx_v.at[b].at[1]], ssems[b]).wait()

    def body(j, carry):
        @pl.when(j > 0)
        def _():
            for b in range(2):
                s_wait(b)
        gathers = [
            pltpu.async_copy(y_hbm.at[idx_v.at[2 * j + b].at[0]],
                             rows_v.at[b], gsems[b])
            for b in range(2)
        ]
        for b in range(2):
            gathers[b].wait()
            pltpu.async_copy(rows_v.at[b],
                             acc_sh.at[idx_v.at[2 * j + b].at[1]],
                             ssems[b], add=True)
        return carry

    lax.fori_loop(0, HALF_CHUNKS // 2, body, 0)
    for b in range(2):
        s_wait(b)
    pltpu.sync_copy(idxc_hbm.at[c].at[pl.ds(base + HALF_CHUNKS, HALF_CHUNKS)], idx_v)
    lax.fori_loop(0, HALF_CHUNKS // 2, body, 0)
    for b in range(2):
        s_wait(b)

    plsc.subcore_barrier()

    r0 = s * ROWS_PER_TILE
    pltpu.sync_copy(
        acc_sh.at[pl.ds(r0, ROWS_PER_TILE)],
        out_hbm.at[c].at[pl.ds(r0, ROWS_PER_TILE)],
    )


@functools.lru_cache(maxsize=1)
def _make_sc_agg_kernel():
    return pl.kernel(
        _sc_agg_body,
        out_type=jax.ShapeDtypeStruct((NUM_CORES, N_PAD, HALF), jnp.float32),
        mesh=plsc.VectorSubcoreMesh(
            core_axis_name="c", subcore_axis_name="s",
            num_cores=NUM_CORES, num_subcores=NUM_TILES,
        ),
        scratch_types=[
            pltpu.VMEM((HALF_CHUNKS, 2, CHUNK), jnp.int32),
            pltpu.VMEM((2, CHUNK, HALF), jnp.float32),
            pltpu.VMEM_SHARED((N_PAD, HALF), jnp.float32),
        ] + [pltpu.SemaphoreType.DMA] * 4,
    )


def _sc_agg(y, idxc, zeros):
    yf = y.reshape(NUM_CORES * N_NODES, HALF)
    return _make_sc_agg_kernel()(yf, idxc, zeros)


def kernel(x, edge_index, W1, b1, W2, b2, W3, b3):
    ei = edge_index.astype(jnp.int32)
    src = ei[0]
    dst = ei[1]

    pad = E_PAD - N_EDGES
    src_p = jnp.concatenate([src, jnp.zeros((pad,), jnp.int32)])
    dst_p = jnp.concatenate([dst, jnp.full((pad,), N_NODES, jnp.int32)])
    srcc = src_p.reshape(N_CHUNKS, CHUNK)
    dstc = dst_p.reshape(N_CHUNKS, CHUNK)
    idxc = jnp.stack([
        jnp.stack([srcc, dstc], axis=1),
        jnp.stack([srcc + N_NODES, dstc], axis=1),
    ])
    zeros = jnp.zeros((ROWS_PER_TILE, HALF), jnp.float32)

    b1h = b1.reshape(NUM_CORES, 1, HALF)
    b2h = b2.reshape(NUM_CORES, 1, HALF)
    b3h = b3.reshape(NUM_CORES, 1, HALF)

    y1 = _mm_first(x, W1)
    a1 = _sc_agg(y1, idxc, zeros)
    y2 = _mm_mid(y1, a1, b1h, W2)
    a2 = _sc_agg(y2, idxc, zeros)
    y3 = _mm_mid(y2, a2, b2h, W3)
    a3 = _sc_agg(y3, idxc, zeros)
    return _final(y3, a3, b3h)
